# baseline (speedup 1.0000x reference)
import jax
import jax.numpy as jnp
from jax.experimental import pallas as pl
from jax.experimental.pallas import tpu as pltpu

_PACK = 4


def _mlp_packed_kernel(xp_ref, xhp_ref, wx_ref, wxh_ref, w2_ref, w3_ref,
                       w4_ref, b1_ref, b2_ref, b3_ref, b4_ref, o_ref):
    h = jnp.dot(xp_ref[...], wx_ref[...], preferred_element_type=jnp.float32)
    h = h + jnp.dot(xhp_ref[...], wxh_ref[...],
                    preferred_element_type=jnp.float32)
    h = jax.nn.sigmoid(h + b1_ref[...])
    h = jax.nn.sigmoid(
        jnp.dot(h, w2_ref[...], preferred_element_type=jnp.float32)
        + b2_ref[...])
    h = jax.nn.sigmoid(
        jnp.dot(h, w3_ref[...], preferred_element_type=jnp.float32)
        + b3_ref[...])
    h = jax.nn.sigmoid(
        jnp.dot(h, w4_ref[...], preferred_element_type=jnp.float32)
        + b4_ref[...])
    o_ref[...] = h.astype(o_ref.dtype)


def kernel(x, x_hat, w1, b1, w2, b2, w3, b3, w4, b4, *, batch_tile=256):
    bs = x.shape[0]
    feat = x.shape[1] * x.shape[2]
    fp = _PACK * feat

    rows = bs // _PACK
    xf = jax.lax.optimization_barrier(x.reshape(bs, feat).astype(jnp.float32))
    xhf = jax.lax.optimization_barrier(
        x_hat.reshape(bs, feat).astype(jnp.float32))
    xp = xf.reshape(rows, fp)
    xhp = xhf.reshape(rows, fp)

    eye = jnp.eye(_PACK, dtype=jnp.float32)
    w1f = w1.astype(jnp.float32)
    wx = jnp.kron(eye, w1f[:feat])
    wxh = jnp.kron(eye, w1f[feat:])
    w2p = jnp.kron(eye, w2.astype(jnp.float32))
    w3p = jnp.kron(eye, w3.astype(jnp.float32))
    w4p = jnp.kron(eye, w4.astype(jnp.float32))
    b1p = jnp.tile(b1.astype(jnp.float32).reshape(1, -1), (1, _PACK))
    b2p = jnp.tile(b2.astype(jnp.float32).reshape(1, -1), (1, _PACK))
    b3p = jnp.tile(b3.astype(jnp.float32).reshape(1, -1), (1, _PACK))
    b4p = jnp.tile(b4.astype(jnp.float32).reshape(1, -1), (1, _PACK))

    tb = min(batch_tile, rows)
    pad = (-rows) % tb
    if pad:
        xp = jnp.pad(xp, ((0, pad), (0, 0)))
        xhp = jnp.pad(xhp, ((0, pad), (0, 0)))
    rows_p = rows + pad
    grid = rows_p // tb

    out = pl.pallas_call(
        _mlp_packed_kernel,
        out_shape=jax.ShapeDtypeStruct((rows_p, _PACK), jnp.float32),
        grid=(grid,),
        in_specs=[
            pl.BlockSpec((tb, fp), lambda i: (i, 0)),
            pl.BlockSpec((tb, fp), lambda i: (i, 0)),
            pl.BlockSpec(wx.shape, lambda i: (0, 0)),
            pl.BlockSpec(wxh.shape, lambda i: (0, 0)),
            pl.BlockSpec(w2p.shape, lambda i: (0, 0)),
            pl.BlockSpec(w3p.shape, lambda i: (0, 0)),
            pl.BlockSpec(w4p.shape, lambda i: (0, 0)),
            pl.BlockSpec(b1p.shape, lambda i: (0, 0)),
            pl.BlockSpec(b2p.shape, lambda i: (0, 0)),
            pl.BlockSpec(b3p.shape, lambda i: (0, 0)),
            pl.BlockSpec(b4p.shape, lambda i: (0, 0)),
        ],
        out_specs=pl.BlockSpec((tb, _PACK), lambda i: (i, 0)),
        compiler_params=pltpu.CompilerParams(
            dimension_semantics=("parallel",)),
    )(xp, xhp, wx, wxh, w2p, w3p, w4p, b1p, b2p, b3p, b4p)

    return out[:rows].reshape(bs, 1)


# speedup vs baseline: 1.2045x; 1.2045x over previous
import jax
import jax.numpy as jnp
from jax.experimental import pallas as pl
from jax.experimental.pallas import tpu as pltpu

_PACK = 4


def _mlp_packed_kernel(xp_ref, xhp_ref, wx_ref, wxh_ref, w2_ref, w3_ref,
                       w4_ref, b1_ref, b2_ref, b3_ref, b4_ref, o_ref):
    h = jnp.dot(xp_ref[...], wx_ref[...], preferred_element_type=jnp.float32)
    h = h + jnp.dot(xhp_ref[...], wxh_ref[...],
                    preferred_element_type=jnp.float32)
    h = jax.nn.sigmoid(h + b1_ref[...])
    h = jax.nn.sigmoid(
        jnp.dot(h, w2_ref[...], preferred_element_type=jnp.float32)
        + b2_ref[...])
    h = jax.nn.sigmoid(
        jnp.dot(h, w3_ref[...], preferred_element_type=jnp.float32)
        + b3_ref[...])
    h = jax.nn.sigmoid(
        jnp.dot(h, w4_ref[...], preferred_element_type=jnp.float32)
        + b4_ref[...])
    o_ref[...] = h.astype(o_ref.dtype)


def kernel(x, x_hat, w1, b1, w2, b2, w3, b3, w4, b4, *, batch_tile=1024):
    bs = x.shape[0]
    feat = x.shape[1] * x.shape[2]
    fp = _PACK * feat

    rows = bs // _PACK
    xf = jax.lax.optimization_barrier(x.reshape(bs, feat).astype(jnp.float32))
    xhf = jax.lax.optimization_barrier(
        x_hat.reshape(bs, feat).astype(jnp.float32))
    xp = xf.reshape(rows, fp)
    xhp = xhf.reshape(rows, fp)

    eye = jnp.eye(_PACK, dtype=jnp.float32)
    w1f = w1.astype(jnp.float32)
    wx = jnp.kron(eye, w1f[:feat])
    wxh = jnp.kron(eye, w1f[feat:])
    w2p = jnp.kron(eye, w2.astype(jnp.float32))
    w3p = jnp.kron(eye, w3.astype(jnp.float32))
    w4p = jnp.kron(eye, w4.astype(jnp.float32))
    b1p = jnp.tile(b1.astype(jnp.float32).reshape(1, -1), (1, _PACK))
    b2p = jnp.tile(b2.astype(jnp.float32).reshape(1, -1), (1, _PACK))
    b3p = jnp.tile(b3.astype(jnp.float32).reshape(1, -1), (1, _PACK))
    b4p = jnp.tile(b4.astype(jnp.float32).reshape(1, -1), (1, _PACK))

    tb = min(batch_tile, rows)
    pad = (-rows) % tb
    if pad:
        xp = jnp.pad(xp, ((0, pad), (0, 0)))
        xhp = jnp.pad(xhp, ((0, pad), (0, 0)))
    rows_p = rows + pad
    grid = rows_p // tb

    out = pl.pallas_call(
        _mlp_packed_kernel,
        out_shape=jax.ShapeDtypeStruct((rows_p, _PACK), jnp.float32),
        grid=(grid,),
        in_specs=[
            pl.BlockSpec((tb, fp), lambda i: (i, 0)),
            pl.BlockSpec((tb, fp), lambda i: (i, 0)),
            pl.BlockSpec(wx.shape, lambda i: (0, 0)),
            pl.BlockSpec(wxh.shape, lambda i: (0, 0)),
            pl.BlockSpec(w2p.shape, lambda i: (0, 0)),
            pl.BlockSpec(w3p.shape, lambda i: (0, 0)),
            pl.BlockSpec(w4p.shape, lambda i: (0, 0)),
            pl.BlockSpec(b1p.shape, lambda i: (0, 0)),
            pl.BlockSpec(b2p.shape, lambda i: (0, 0)),
            pl.BlockSpec(b3p.shape, lambda i: (0, 0)),
            pl.BlockSpec(b4p.shape, lambda i: (0, 0)),
        ],
        out_specs=pl.BlockSpec((tb, _PACK), lambda i: (i, 0)),
        compiler_params=pltpu.CompilerParams(
            dimension_semantics=("parallel",)),
    )(xp, xhp, wx, wxh, w2p, w3p, w4p, b1p, b2p, b3p, b4p)

    return out[:rows].reshape(bs, 1)


# speedup vs baseline: 1.2320x; 1.0228x over previous
import jax
import jax.numpy as jnp
from jax.experimental import pallas as pl
from jax.experimental.pallas import tpu as pltpu

_PACK = 4


def _mlp_packed_kernel(xp_ref, xhp_ref, wx_ref, wxh_ref, w2_ref, w3_ref,
                       w4_ref, b1_ref, b2_ref, b3_ref, b4_ref, o_ref):
    h = jnp.dot(xp_ref[...], wx_ref[...], preferred_element_type=jnp.float32)
    h = h + jnp.dot(xhp_ref[...], wxh_ref[...],
                    preferred_element_type=jnp.float32)
    h = jax.nn.sigmoid(h + b1_ref[...])
    h = jax.nn.sigmoid(
        jnp.dot(h, w2_ref[...], preferred_element_type=jnp.float32)
        + b2_ref[...])
    h = jax.nn.sigmoid(
        jnp.dot(h, w3_ref[...], preferred_element_type=jnp.float32)
        + b3_ref[...])
    h = jax.nn.sigmoid(
        jnp.dot(h, w4_ref[...], preferred_element_type=jnp.float32)
        + b4_ref[...])
    o_ref[...] = h.astype(o_ref.dtype)


def kernel(x, x_hat, w1, b1, w2, b2, w3, b3, w4, b4, *, batch_tile=2048):
    bs = x.shape[0]
    feat = x.shape[1] * x.shape[2]
    fp = _PACK * feat

    rows = bs // _PACK
    xf = jax.lax.optimization_barrier(x.reshape(bs, feat).astype(jnp.float32))
    xhf = jax.lax.optimization_barrier(
        x_hat.reshape(bs, feat).astype(jnp.float32))
    xp = xf.reshape(rows, fp)
    xhp = xhf.reshape(rows, fp)

    eye = jnp.eye(_PACK, dtype=jnp.float32)
    w1f = w1.astype(jnp.float32)
    wx = jnp.kron(eye, w1f[:feat])
    wxh = jnp.kron(eye, w1f[feat:])
    w2p = jnp.kron(eye, w2.astype(jnp.float32))
    w3p = jnp.kron(eye, w3.astype(jnp.float32))
    w4p = jnp.kron(eye, w4.astype(jnp.float32))
    b1p = jnp.tile(b1.astype(jnp.float32).reshape(1, -1), (1, _PACK))
    b2p = jnp.tile(b2.astype(jnp.float32).reshape(1, -1), (1, _PACK))
    b3p = jnp.tile(b3.astype(jnp.float32).reshape(1, -1), (1, _PACK))
    b4p = jnp.tile(b4.astype(jnp.float32).reshape(1, -1), (1, _PACK))

    tb = min(batch_tile, rows)
    pad = (-rows) % tb
    if pad:
        xp = jnp.pad(xp, ((0, pad), (0, 0)))
        xhp = jnp.pad(xhp, ((0, pad), (0, 0)))
    rows_p = rows + pad
    grid = rows_p // tb

    out = pl.pallas_call(
        _mlp_packed_kernel,
        out_shape=jax.ShapeDtypeStruct((rows_p, _PACK), jnp.float32),
        grid=(grid,),
        in_specs=[
            pl.BlockSpec((tb, fp), lambda i: (i, 0)),
            pl.BlockSpec((tb, fp), lambda i: (i, 0)),
            pl.BlockSpec(wx.shape, lambda i: (0, 0)),
            pl.BlockSpec(wxh.shape, lambda i: (0, 0)),
            pl.BlockSpec(w2p.shape, lambda i: (0, 0)),
            pl.BlockSpec(w3p.shape, lambda i: (0, 0)),
            pl.BlockSpec(w4p.shape, lambda i: (0, 0)),
            pl.BlockSpec(b1p.shape, lambda i: (0, 0)),
            pl.BlockSpec(b2p.shape, lambda i: (0, 0)),
            pl.BlockSpec(b3p.shape, lambda i: (0, 0)),
            pl.BlockSpec(b4p.shape, lambda i: (0, 0)),
        ],
        out_specs=pl.BlockSpec((tb, _PACK), lambda i: (i, 0)),
        compiler_params=pltpu.CompilerParams(
            dimension_semantics=("parallel",)),
    )(xp, xhp, wx, wxh, w2p, w3p, w4p, b1p, b2p, b3p, b4p)

    return out[:rows].reshape(bs, 1)


# speedup vs baseline: 1.6054x; 1.3030x over previous
import jax
import jax.numpy as jnp
from jax.experimental import pallas as pl
from jax.experimental.pallas import tpu as pltpu


def _mlp_t_kernel(x_ref, xh_ref, w1x_ref, w1h_ref, w2_ref, w3_ref, w4_ref,
                  b1_ref, b2_ref, b3_ref, b4_ref, o_ref):
    ct = (((1,), (1,)), ((), ()))
    h = jax.lax.dot_general(w1x_ref[...], x_ref[...], ct,
                            preferred_element_type=jnp.float32)
    h = h + jax.lax.dot_general(w1h_ref[...], xh_ref[...], ct,
                                preferred_element_type=jnp.float32)
    h = jax.nn.sigmoid(h + b1_ref[...])
    h = jax.nn.sigmoid(
        jnp.dot(w2_ref[...], h, preferred_element_type=jnp.float32)
        + b2_ref[...])
    h = jax.nn.sigmoid(
        jnp.dot(w3_ref[...], h, preferred_element_type=jnp.float32)
        + b3_ref[...])
    h = jax.nn.sigmoid(
        jnp.dot(w4_ref[...], h, preferred_element_type=jnp.float32)
        + b4_ref[...])
    o_ref[...] = h[0:1, :].astype(o_ref.dtype)


def kernel(x, x_hat, w1, b1, w2, b2, w3, b3, w4, b4, *, batch_tile=4096):
    bs = x.shape[0]
    feat = x.shape[1] * x.shape[2]

    xf = x.reshape(bs, feat).astype(jnp.float32)
    xhf = x_hat.reshape(bs, feat).astype(jnp.float32)

    w1f = w1.astype(jnp.float32)
    w1x = w1f[:feat].T
    w1h = w1f[feat:].T
    w2t = w2.astype(jnp.float32).T
    w3t = w3.astype(jnp.float32).T
    w4t = jnp.zeros((8, 16), jnp.float32).at[0:1, :].set(
        w4.astype(jnp.float32).T)
    b1c = b1.astype(jnp.float32).reshape(-1, 1)
    b2c = b2.astype(jnp.float32).reshape(-1, 1)
    b3c = b3.astype(jnp.float32).reshape(-1, 1)
    b4c = jnp.zeros((8, 1), jnp.float32).at[0:1, :].set(
        b4.astype(jnp.float32).reshape(1, 1))

    bt = min(batch_tile, bs)
    pad = (-bs) % bt
    if pad:
        xf = jnp.pad(xf, ((0, pad), (0, 0)))
        xhf = jnp.pad(xhf, ((0, pad), (0, 0)))
    bs_p = bs + pad
    grid = bs_p // bt

    out = pl.pallas_call(
        _mlp_t_kernel,
        out_shape=jax.ShapeDtypeStruct((1, bs_p), jnp.float32),
        grid=(grid,),
        in_specs=[
            pl.BlockSpec((bt, feat), lambda i: (i, 0)),
            pl.BlockSpec((bt, feat), lambda i: (i, 0)),
            pl.BlockSpec(w1x.shape, lambda i: (0, 0)),
            pl.BlockSpec(w1h.shape, lambda i: (0, 0)),
            pl.BlockSpec(w2t.shape, lambda i: (0, 0)),
            pl.BlockSpec(w3t.shape, lambda i: (0, 0)),
            pl.BlockSpec(w4t.shape, lambda i: (0, 0)),
            pl.BlockSpec(b1c.shape, lambda i: (0, 0)),
            pl.BlockSpec(b2c.shape, lambda i: (0, 0)),
            pl.BlockSpec(b3c.shape, lambda i: (0, 0)),
            pl.BlockSpec(b4c.shape, lambda i: (0, 0)),
        ],
        out_specs=pl.BlockSpec((1, bt), lambda i: (0, i)),
        compiler_params=pltpu.CompilerParams(
            dimension_semantics=("parallel",)),
    )(xf, xhf, w1x, w1h, w2t, w3t, w4t, b1c, b2c, b3c, b4c)

    return out[0, :bs].reshape(bs, 1)


# speedup vs baseline: 2.2651x; 1.4109x over previous
import jax
import jax.numpy as jnp
from jax.experimental import pallas as pl
from jax.experimental.pallas import tpu as pltpu


def _mlp_t_kernel(x_ref, xh_ref, w1x_ref, w1h_ref, w2_ref, w3_ref, w4_ref,
                  b1_ref, b2_ref, b3_ref, b4_ref, o_ref):
    h = jnp.dot(w1x_ref[...], x_ref[...],
                preferred_element_type=jnp.float32)
    h = h + jnp.dot(w1h_ref[...], xh_ref[...],
                    preferred_element_type=jnp.float32)
    h = jax.nn.sigmoid(h + b1_ref[...])
    h = jax.nn.sigmoid(
        jnp.dot(w2_ref[...], h, preferred_element_type=jnp.float32)
        + b2_ref[...])
    h = jax.nn.sigmoid(
        jnp.dot(w3_ref[...], h, preferred_element_type=jnp.float32)
        + b3_ref[...])
    h = jax.nn.sigmoid(
        jnp.dot(w4_ref[...], h, preferred_element_type=jnp.float32)
        + b4_ref[...])
    o_ref[...] = h[0:1, :].astype(o_ref.dtype)


def kernel(x, x_hat, w1, b1, w2, b2, w3, b3, w4, b4, *, batch_tile=4096):
    bs = x.shape[0]
    feat = x.shape[1] * x.shape[2]

    xf = x.transpose(1, 2, 0).reshape(feat, bs).astype(jnp.float32)
    xhf = x_hat.transpose(1, 2, 0).reshape(feat, bs).astype(jnp.float32)

    w1f = w1.astype(jnp.float32)
    w1x = w1f[:feat].T
    w1h = w1f[feat:].T
    w2t = w2.astype(jnp.float32).T
    w3t = w3.astype(jnp.float32).T
    w4t = jnp.zeros((8, 16), jnp.float32).at[0:1, :].set(
        w4.astype(jnp.float32).T)
    b1c = b1.astype(jnp.float32).reshape(-1, 1)
    b2c = b2.astype(jnp.float32).reshape(-1, 1)
    b3c = b3.astype(jnp.float32).reshape(-1, 1)
    b4c = jnp.zeros((8, 1), jnp.float32).at[0:1, :].set(
        b4.astype(jnp.float32).reshape(1, 1))

    bt = min(batch_tile, bs)
    pad = (-bs) % bt
    if pad:
        xf = jnp.pad(xf, ((0, 0), (0, pad)))
        xhf = jnp.pad(xhf, ((0, 0), (0, pad)))
    bs_p = bs + pad
    grid = bs_p // bt

    out = pl.pallas_call(
        _mlp_t_kernel,
        out_shape=jax.ShapeDtypeStruct((1, bs_p), jnp.float32),
        grid=(grid,),
        in_specs=[
            pl.BlockSpec((feat, bt), lambda i: (0, i)),
            pl.BlockSpec((feat, bt), lambda i: (0, i)),
            pl.BlockSpec(w1x.shape, lambda i: (0, 0)),
            pl.BlockSpec(w1h.shape, lambda i: (0, 0)),
            pl.BlockSpec(w2t.shape, lambda i: (0, 0)),
            pl.BlockSpec(w3t.shape, lambda i: (0, 0)),
            pl.BlockSpec(w4t.shape, lambda i: (0, 0)),
            pl.BlockSpec(b1c.shape, lambda i: (0, 0)),
            pl.BlockSpec(b2c.shape, lambda i: (0, 0)),
            pl.BlockSpec(b3c.shape, lambda i: (0, 0)),
            pl.BlockSpec(b4c.shape, lambda i: (0, 0)),
        ],
        out_specs=pl.BlockSpec((1, bt), lambda i: (0, i)),
        compiler_params=pltpu.CompilerParams(
            dimension_semantics=("parallel",)),
    )(xf, xhf, w1x, w1h, w2t, w3t, w4t, b1c, b2c, b3c, b4c)

    return out[0, :bs].reshape(bs, 1)


# speedup vs baseline: 2.3443x; 1.0350x over previous
import jax
import jax.numpy as jnp
from jax.experimental import pallas as pl
from jax.experimental.pallas import tpu as pltpu


def _mlp_t_kernel(x_ref, xh_ref, w1x_ref, w1h_ref, w2_ref, w3_ref, w4_ref,
                  b1_ref, b2_ref, b3_ref, b4_ref, o_ref):
    h = jnp.dot(w1x_ref[...], x_ref[...],
                preferred_element_type=jnp.float32)
    h = h + jnp.dot(w1h_ref[...], xh_ref[...],
                    preferred_element_type=jnp.float32)
    h = jax.nn.sigmoid(h + b1_ref[...])
    h = jax.nn.sigmoid(
        jnp.dot(w2_ref[...], h, preferred_element_type=jnp.float32)
        + b2_ref[...])
    h = jax.nn.sigmoid(
        jnp.dot(w3_ref[...], h, preferred_element_type=jnp.float32)
        + b3_ref[...])
    h = jax.nn.sigmoid(
        jnp.dot(w4_ref[...], h, preferred_element_type=jnp.float32)
        + b4_ref[...])
    o_ref[...] = h[0:1, :].astype(o_ref.dtype)


def kernel(x, x_hat, w1, b1, w2, b2, w3, b3, w4, b4, *, batch_tile=8192):
    bs = x.shape[0]
    feat = x.shape[1] * x.shape[2]

    xf = x.transpose(1, 2, 0).reshape(feat, bs).astype(jnp.float32)
    xhf = x_hat.transpose(1, 2, 0).reshape(feat, bs).astype(jnp.float32)

    w1f = w1.astype(jnp.float32)
    w1x = w1f[:feat].T
    w1h = w1f[feat:].T
    w2t = w2.astype(jnp.float32).T
    w3t = w3.astype(jnp.float32).T
    w4t = jnp.zeros((8, 16), jnp.float32).at[0:1, :].set(
        w4.astype(jnp.float32).T)
    b1c = b1.astype(jnp.float32).reshape(-1, 1)
    b2c = b2.astype(jnp.float32).reshape(-1, 1)
    b3c = b3.astype(jnp.float32).reshape(-1, 1)
    b4c = jnp.zeros((8, 1), jnp.float32).at[0:1, :].set(
        b4.astype(jnp.float32).reshape(1, 1))

    bt = min(batch_tile, bs)
    pad = (-bs) % bt
    if pad:
        xf = jnp.pad(xf, ((0, 0), (0, pad)))
        xhf = jnp.pad(xhf, ((0, 0), (0, pad)))
    bs_p = bs + pad
    grid = bs_p // bt

    out = pl.pallas_call(
        _mlp_t_kernel,
        out_shape=jax.ShapeDtypeStruct((1, bs_p), jnp.float32),
        grid=(grid,),
        in_specs=[
            pl.BlockSpec((feat, bt), lambda i: (0, i)),
            pl.BlockSpec((feat, bt), lambda i: (0, i)),
            pl.BlockSpec(w1x.shape, lambda i: (0, 0)),
            pl.BlockSpec(w1h.shape, lambda i: (0, 0)),
            pl.BlockSpec(w2t.shape, lambda i: (0, 0)),
            pl.BlockSpec(w3t.shape, lambda i: (0, 0)),
            pl.BlockSpec(w4t.shape, lambda i: (0, 0)),
            pl.BlockSpec(b1c.shape, lambda i: (0, 0)),
            pl.BlockSpec(b2c.shape, lambda i: (0, 0)),
            pl.BlockSpec(b3c.shape, lambda i: (0, 0)),
            pl.BlockSpec(b4c.shape, lambda i: (0, 0)),
        ],
        out_specs=pl.BlockSpec((1, bt), lambda i: (0, i)),
        compiler_params=pltpu.CompilerParams(
            dimension_semantics=("parallel",)),
    )(xf, xhf, w1x, w1h, w2t, w3t, w4t, b1c, b2c, b3c, b4c)

    return out[0, :bs].reshape(bs, 1)


# speedup vs baseline: 2.4347x; 1.0386x over previous
import jax
import jax.numpy as jnp
from jax.experimental import pallas as pl
from jax.experimental.pallas import tpu as pltpu


def _mlp_t_kernel(x_ref, xh_ref, w1x_ref, w1h_ref, w2_ref, w3_ref, w4_ref,
                  b1_ref, b2_ref, b3_ref, b4_ref, o_ref):
    def sig(v):
        return 0.5 * jnp.tanh(0.5 * v) + 0.5

    h = jnp.dot(w1x_ref[...], x_ref[...],
                preferred_element_type=jnp.float32)
    h = h + jnp.dot(w1h_ref[...], xh_ref[...],
                    preferred_element_type=jnp.float32)
    h = sig(h + b1_ref[...])
    h = sig(
        jnp.dot(w2_ref[...], h, preferred_element_type=jnp.float32)
        + b2_ref[...])
    h = sig(
        jnp.dot(w3_ref[...], h, preferred_element_type=jnp.float32)
        + b3_ref[...])
    h = jnp.dot(w4_ref[...], h, preferred_element_type=jnp.float32)
    h = sig(h[0:1, :] + b4_ref[0:1, :])
    o_ref[...] = h.astype(o_ref.dtype)


def kernel(x, x_hat, w1, b1, w2, b2, w3, b3, w4, b4, *, batch_tile=8192):
    bs = x.shape[0]
    feat = x.shape[1] * x.shape[2]

    xf = x.transpose(1, 2, 0).reshape(feat, bs).astype(jnp.float32)
    xhf = x_hat.transpose(1, 2, 0).reshape(feat, bs).astype(jnp.float32)

    w1f = w1.astype(jnp.float32)
    w1x = w1f[:feat].T
    w1h = w1f[feat:].T
    w2t = w2.astype(jnp.float32).T
    w3t = w3.astype(jnp.float32).T
    w4t = jnp.zeros((8, 16), jnp.float32).at[0:1, :].set(
        w4.astype(jnp.float32).T)
    b1c = b1.astype(jnp.float32).reshape(-1, 1)
    b2c = b2.astype(jnp.float32).reshape(-1, 1)
    b3c = b3.astype(jnp.float32).reshape(-1, 1)
    b4c = jnp.zeros((8, 1), jnp.float32).at[0:1, :].set(
        b4.astype(jnp.float32).reshape(1, 1))

    bt = min(batch_tile, bs)
    pad = (-bs) % bt
    if pad:
        xf = jnp.pad(xf, ((0, 0), (0, pad)))
        xhf = jnp.pad(xhf, ((0, 0), (0, pad)))
    bs_p = bs + pad
    grid = bs_p // bt

    out = pl.pallas_call(
        _mlp_t_kernel,
        out_shape=jax.ShapeDtypeStruct((1, bs_p), jnp.float32),
        grid=(grid,),
        in_specs=[
            pl.BlockSpec((feat, bt), lambda i: (0, i)),
            pl.BlockSpec((feat, bt), lambda i: (0, i)),
            pl.BlockSpec(w1x.shape, lambda i: (0, 0)),
            pl.BlockSpec(w1h.shape, lambda i: (0, 0)),
            pl.BlockSpec(w2t.shape, lambda i: (0, 0)),
            pl.BlockSpec(w3t.shape, lambda i: (0, 0)),
            pl.BlockSpec(w4t.shape, lambda i: (0, 0)),
            pl.BlockSpec(b1c.shape, lambda i: (0, 0)),
            pl.BlockSpec(b2c.shape, lambda i: (0, 0)),
            pl.BlockSpec(b3c.shape, lambda i: (0, 0)),
            pl.BlockSpec(b4c.shape, lambda i: (0, 0)),
        ],
        out_specs=pl.BlockSpec((1, bt), lambda i: (0, i)),
        compiler_params=pltpu.CompilerParams(
            dimension_semantics=("parallel",)),
    )(xf, xhf, w1x, w1h, w2t, w3t, w4t, b1c, b2c, b3c, b4c)

    return out[0, :bs].reshape(bs, 1)
